# Initial kernel scaffold; baseline (speedup 1.0000x reference)
import jax
import jax.numpy as jnp
from jax.experimental import pallas as pl


def kernel(x, conv1_w, conv1_b, score_w, score_b, loc_w, loc_b, img_size):
    raise NotImplementedError("write your pallas kernel here")


# speedup vs baseline: 13.7537x; 13.7537x over previous
import numpy as np
import jax
import jax.numpy as jnp
from jax.experimental import pallas as pl
from jax.experimental.pallas import tpu as pltpu

_FULL = 2704
_WIDE = 2816
_EXT = 2944
_NANCH = 22500
_NPAD = 22528
_ROWS = 176
_PRE_K = 12000
_POST_K = 2000
_OUT_ROWS = 2048
_NEG = -1e30


def _conv3x3_body(x_ref, w_ref, y_ref):
    k = pl.program_id(1)

    @pl.when(k == 0)
    def _():
        y_ref[...] = jnp.zeros_like(y_ref)

    y_ref[...] += jnp.dot(w_ref[...], x_ref[...],
                          preferred_element_type=jnp.float32)


def _heads_body(y_ref, b1_ref, wsl_ref, bsl_ref, sl_ref):
    h = jnp.maximum(y_ref[...] + b1_ref[...], 0.0)
    sl_ref[...] = jnp.dot(wsl_ref[...], h,
                          preferred_element_type=jnp.float32) + bsl_ref[...]


def _proposal_body(loc_ref, anc_ref, sc_ref, out_ref):
    l0, l1, l2, l3 = (loc_ref[i] for i in range(4))
    a0, a1, a2, a3 = (anc_ref[i] for i in range(4))

    wa = a2 - a0
    ha = a3 - a1
    cxa = (a0 + a2) / 2.0
    cya = (a1 + a3) / 2.0
    cx = l0 * wa / 10.0 + cxa
    cy = l1 * ha / 10.0 + cya
    w = jnp.exp(l2 / 5.0) * wa
    h = jnp.exp(l3 / 5.0) * ha
    x1 = jnp.clip(cx - w / 2.0, 0.0, 1.0)
    y1 = jnp.clip(cy - h / 2.0, 0.0, 1.0)
    x2 = jnp.clip(cx + w / 2.0, 0.0, 1.0)
    y2 = jnp.clip(cy + h / 2.0, 0.0, 1.0)
    ws = x2 - x1
    hs = y2 - y1
    areas = ws * hs

    s0 = sc_ref[0]
    s1 = sc_ref[1]
    m01 = jnp.maximum(s0, s1)
    e0 = jnp.exp(s0 - m01)
    e1 = jnp.exp(s1 - m01)
    fg = e1 / (e0 + e1)

    rows = jax.lax.broadcasted_iota(jnp.int32, (_ROWS, 128), 0)
    cols = jax.lax.broadcasted_iota(jnp.int32, (_ROWS, 128), 1)
    gidx = rows * 128 + cols
    in_range = gidx < _NANCH

    size_ok = (hs >= 16.0 / 1000.0) & (ws >= 16.0 / 1000.0)
    score = jnp.where(size_ok & in_range, fg, _NEG)

    bits = jax.lax.bitcast_convert_type(score, jnp.uint32)
    sign = (bits >> jnp.uint32(31)).astype(jnp.bool_)
    u = jnp.where(sign, ~bits, bits | jnp.uint32(0x80000000))

    def _tstep(i, t):
        cand = t | (jnp.uint32(0x80000000) >> i.astype(jnp.uint32))
        cnt = jnp.sum((u >= cand).astype(jnp.int32))
        return jnp.where(cnt >= _PRE_K, cand, t)

    thresh = jax.lax.fori_loop(0, 32, _tstep, jnp.uint32(0))
    sc0 = jnp.where(u >= thresh, score, _NEG)

    big = jnp.int32(1 << 30)
    lane = jax.lax.broadcasted_iota(jnp.int32, (1, 128), 1)

    def _best_box(sc):
        m = jnp.max(sc)
        idx = jnp.min(jnp.where(sc == m, gidx, big))
        is_best = gidx == idx
        zero = jnp.zeros_like(sc)
        bx1 = jnp.sum(jnp.where(is_best, x1, zero))
        by1 = jnp.sum(jnp.where(is_best, y1, zero))
        bx2 = jnp.sum(jnp.where(is_best, x2, zero))
        by2 = jnp.sum(jnp.where(is_best, y2, zero))
        return m, is_best, bx1, by1, bx2, by2

    def _box_row(bx1, by1, bx2, by2):
        r = jnp.where(lane == 0, bx1, 0.0)
        r = jnp.where(lane == 1, by1, r)
        r = jnp.where(lane == 2, bx2, r)
        r = jnp.where(lane == 3, by2, r)
        return r

    _, _, px1, py1, px2, py2 = _best_box(sc0)
    out_ref[...] = jnp.broadcast_to(_box_row(px1, py1, px2, py2),
                                    (_OUT_ROWS, 128))

    def _cond(carry):
        k, cont, _ = carry
        return (k < _POST_K) & cont

    def _body(carry):
        k, _, sc = carry
        m, is_best, bx1, by1, bx2, by2 = _best_box(sc)
        valid = m > -1e20
        barea = (bx2 - bx1) * (by2 - by1)
        xx1 = jnp.maximum(bx1, x1)
        yy1 = jnp.maximum(by1, y1)
        xx2 = jnp.minimum(bx2, x2)
        yy2 = jnp.minimum(by2, y2)
        inter = jnp.clip(xx2 - xx1, 0.0) * jnp.clip(yy2 - yy1, 0.0)
        iou = inter / (barea + areas - inter + 1e-9)
        nsc = jnp.where((iou > 0.7) | is_best, _NEG, sc)

        @pl.when(valid)
        def _():
            out_ref[pl.ds(k, 1), :] = _box_row(bx1, by1, bx2, by2)

        return (k + valid.astype(jnp.int32), valid,
                jnp.where(valid, nsc, sc))

    jax.lax.while_loop(_cond, _body, (jnp.int32(0), True, sc0))


def _anchor_base(base_size=16, ratios=(0.5, 1.0, 2.0), scales=(8, 16, 32)):
    px = base_size / 2.0
    py = base_size / 2.0
    anchors = []
    for r in ratios:
        for s in scales:
            h = base_size * s * np.sqrt(r)
            w = base_size * s * np.sqrt(1.0 / r)
            anchors.append([px - w / 2.0, py - h / 2.0,
                            px + w / 2.0, py + h / 2.0])
    return np.array(anchors, dtype=np.float32)


def _all_anchors(hh, ww, feat_stride=16):
    shift_x = np.arange(ww) * feat_stride
    shift_y = np.arange(hh) * feat_stride
    sx, sy = np.meshgrid(shift_x, shift_y)
    shifts = np.stack([sx.ravel(), sy.ravel(), sx.ravel(), sy.ravel()],
                      axis=1).astype(np.float32)
    return (shifts[:, None, :] + _anchor_base()[None, :, :]).reshape(-1, 4)


def kernel(x, conv1_w, conv1_b, score_w, score_b, loc_w, loc_b, img_size):
    n, c, hh, ww = x.shape
    anchor = jnp.asarray(_all_anchors(hh, ww)) / jnp.asarray(
        img_size).astype(jnp.float32)

    xpad = jnp.pad(x[0], ((0, 0), (1, 1), (1, 1))).reshape(c, _FULL)
    xe = jnp.pad(xpad, ((0, 0), (53, _EXT - _FULL - 53)))
    xs = jnp.stack([xe[:, (ky * 52 + kx):(ky * 52 + kx) + _WIDE]
                    for ky in range(3) for kx in range(3)])
    x2 = jnp.transpose(xs, (1, 0, 2)).reshape(9 * c, _WIDE)
    w2 = conv1_w.reshape(c, 9 * c)

    half = _WIDE // 2
    y = pl.pallas_call(
        _conv3x3_body,
        grid=(2, 9),
        in_specs=[
            pl.BlockSpec((c, half), lambda j, k: (k, j)),
            pl.BlockSpec((c, c), lambda j, k: (0, k)),
        ],
        out_specs=pl.BlockSpec((c, half), lambda j, k: (0, j)),
        out_shape=jax.ShapeDtypeStruct((c, _WIDE), jnp.float32),
        compiler_params=pltpu.CompilerParams(
            dimension_semantics=("parallel", "arbitrary")),
    )(x2, w2)

    wsl = jnp.zeros((64, c), jnp.float32)
    wsl = wsl.at[:18].set(score_w[:, :, 0, 0]).at[18:54].set(loc_w[:, :, 0, 0])
    bsl = jnp.zeros((64,), jnp.float32)
    bsl = bsl.at[:18].set(score_b).at[18:54].set(loc_b)

    sl = pl.pallas_call(
        _heads_body,
        grid=(2,),
        in_specs=[
            pl.BlockSpec((c, half), lambda j: (0, j)),
            pl.BlockSpec((c, 1), lambda j: (0, 0)),
            pl.BlockSpec((64, c), lambda j: (0, 0)),
            pl.BlockSpec((64, 1), lambda j: (0, 0)),
        ],
        out_specs=pl.BlockSpec((64, half), lambda j: (0, j)),
        out_shape=jax.ShapeDtypeStruct((64, _WIDE), jnp.float32),
    )(y, conv1_b.reshape(c, 1), wsl, bsl.reshape(64, 1))

    slv = sl[:, :_FULL].reshape(64, 52, 52)[:, 1:51, 1:51]
    rpn_scores = slv[:18].transpose(1, 2, 0).reshape(1, _NANCH, 2)
    rpn_locs = slv[18:54].transpose(1, 2, 0).reshape(1, _NANCH, 4)

    pad = _NPAD - _NANCH
    loc4 = jnp.pad(rpn_locs[0].T, ((0, 0), (0, pad))).reshape(4, _ROWS, 128)
    anc4 = jnp.pad(anchor.T, ((0, 0), (0, pad))).reshape(4, _ROWS, 128)
    sc2 = jnp.pad(rpn_scores[0].T, ((0, 0), (0, pad))).reshape(2, _ROWS, 128)

    rois_buf = pl.pallas_call(
        _proposal_body,
        out_shape=jax.ShapeDtypeStruct((_OUT_ROWS, 128), jnp.float32),
    )(loc4, anc4, sc2)

    rois = rois_buf[:_POST_K, :4]
    roi_indices = jnp.zeros((_POST_K,), jnp.int32)
    return (rpn_locs, rpn_scores, rois, roi_indices, anchor)
